# Initial kernel scaffold; baseline (speedup 1.0000x reference)
import jax
import jax.numpy as jnp
from jax.experimental import pallas as pl


def kernel(input_ids, embedding_table, pooler_w, pooler_b):
    raise NotImplementedError("write your pallas kernel here")


# speedup vs baseline: 2.2157x; 2.2157x over previous
import functools

import jax
import jax.numpy as jnp
from jax import lax
from jax.experimental import pallas as pl
from jax.experimental.pallas import tpu as pltpu
from jax.experimental.pallas import tpu_sc as plsc

VOCAB = 1000000
HIDDEN = 64
BATCH = 16384
SEQ = 200

NC = 2
NS = 16
NW = NC * NS

ROWS_PER_TILE = BATCH // NW
CB = 4
NCHUNK = ROWS_PER_TILE // CB
SEG = 100
SEG_PER_ROW = SEQ // SEG
NSEG = CB * SEG_PER_ROW


def _sc_pooled_sums(ids2d, table, didx):
    mesh = plsc.VectorSubcoreMesh(core_axis_name="c", subcore_axis_name="s")

    @functools.partial(
        pl.kernel,
        mesh=mesh,
        compiler_params=pltpu.CompilerParams(use_tc_tiling_on_sc=False),
        out_type=jax.ShapeDtypeStruct((BATCH, HIDDEN), jnp.float32),
        scratch_types=[
            pltpu.VMEM((NSEG, SEG), jnp.int32),
            pltpu.VMEM((NSEG * SEG, HIDDEN), jnp.float32),
            pltpu.VMEM((CB, HIDDEN), jnp.float32),
            pltpu.SemaphoreType.DMA,
        ],
    )
    def k(ids_hbm, table_hbm, didx_hbm, out_hbm,
          ids_v, rows_v, sums_v, sem):
        cid = lax.axis_index("c")
        sid = lax.axis_index("s")
        wid = sid * NC + cid

        seg0 = wid * (NCHUNK * NSEG)
        row0 = wid * ROWS_PER_TILE

        def chunk(c, carry):
            pltpu.sync_copy(ids_hbm.at[pl.ds(seg0 + c * NSEG, NSEG)], ids_v)
            handles = []
            for s in range(NSEG):
                handles.append(pltpu.async_copy(
                    table_hbm.at[ids_v.at[s]],
                    rows_v.at[pl.ds(s * SEG, SEG)],
                    sem))
            for h in handles:
                h.wait()
            for r in range(CB):
                def tok(t, acc):
                    return tuple(
                        acc[g] + rows_v[r * SEQ + t, pl.ds(g * 16, 16)]
                        for g in range(HIDDEN // 16))
                acc0 = tuple(jnp.zeros((16,), jnp.float32)
                             for _ in range(HIDDEN // 16))
                acc = lax.fori_loop(0, SEQ, tok, acc0)
                for g in range(HIDDEN // 16):
                    sums_v[r, pl.ds(g * 16, 16)] = acc[g]
            pltpu.sync_copy(sums_v, out_hbm.at[pl.ds(row0 + c * CB, CB)])
            return carry

        lax.fori_loop(0, NCHUNK, chunk, 0)

    return k(ids2d, table, didx)


def _tc_pooler(sums, a, b):
    bt = 512

    def body(x_ref, a_ref, b_ref, o_ref):
        o_ref[...] = jnp.dot(x_ref[...], a_ref[...],
                             preferred_element_type=jnp.float32) + b_ref[...]

    return pl.pallas_call(
        body,
        grid=(BATCH // bt,),
        in_specs=[
            pl.BlockSpec((bt, HIDDEN), lambda i: (i, 0)),
            pl.BlockSpec((HIDDEN, HIDDEN), lambda i: (0, 0)),
            pl.BlockSpec((1, HIDDEN), lambda i: (0, 0)),
        ],
        out_specs=pl.BlockSpec((bt, HIDDEN), lambda i: (i, 0)),
        out_shape=jax.ShapeDtypeStruct((BATCH, HIDDEN), jnp.float32),
    )(sums, a, b)


def kernel(input_ids, embedding_table, pooler_w, pooler_b):
    ids2d = jnp.reshape(input_ids.astype(jnp.int32), (BATCH * SEQ // SEG, SEG))
    didx = (jnp.arange(NS, dtype=jnp.int32)[:, None, None] * CB
            + (jnp.arange(NSEG, dtype=jnp.int32) // SEG_PER_ROW)[None, :, None]
            + jnp.zeros((SEG,), jnp.int32)[None, None, :])
    sums = _sc_pooled_sums(ids2d, embedding_table, didx)
    a = pooler_w.T * (1.0 / SEQ)
    b2d = jnp.reshape(pooler_b, (1, HIDDEN))
    return _tc_pooler(sums, a, b2d)


# speedup vs baseline: 3.1201x; 1.4082x over previous
import functools

import jax
import jax.numpy as jnp
from jax import lax
from jax.experimental import pallas as pl
from jax.experimental.pallas import tpu as pltpu
from jax.experimental.pallas import tpu_sc as plsc

VOCAB = 1000000
HIDDEN = 64
BATCH = 16384
SEQ = 200
NG = HIDDEN // 16

NC = 2
NS = 16
NW = NC * NS

ROWS_PER_TILE = BATCH // NW
CB = 4
NCHUNK = ROWS_PER_TILE // CB
SEG = 100
NSEG = CB * SEQ // SEG
UNROLL = 4


def _sc_pooled_sums(ids2d, table):
    mesh = plsc.VectorSubcoreMesh(core_axis_name="c", subcore_axis_name="s")

    @functools.partial(
        pl.kernel,
        mesh=mesh,
        compiler_params=pltpu.CompilerParams(use_tc_tiling_on_sc=False),
        out_type=jax.ShapeDtypeStruct((BATCH, HIDDEN), jnp.float32),
        scratch_types=[
            pltpu.VMEM((2, NSEG, SEG), jnp.int32),
            pltpu.VMEM((2, NSEG * SEG, HIDDEN), jnp.float32),
            pltpu.VMEM((2, CB, HIDDEN), jnp.float32),
            pltpu.SemaphoreType.DMA,
            pltpu.SemaphoreType.DMA,
            pltpu.SemaphoreType.DMA,
            pltpu.SemaphoreType.DMA,
        ],
    )
    def k(ids_hbm, table_hbm, out_hbm,
          ids_v, rows_v, sums_v, sem_g0, sem_g1, sem_i, sem_o):
        cid = lax.axis_index("c")
        sid = lax.axis_index("s")
        wid = sid * NC + cid

        seg0 = wid * (NCHUNK * NSEG)
        row0 = wid * ROWS_PER_TILE
        sem_g = (sem_g0, sem_g1)

        def ids_fire(c, b):
            pltpu.async_copy(ids_hbm.at[pl.ds(seg0 + c * NSEG, NSEG)],
                             ids_v.at[b], sem_i)

        def ids_wait(b):
            pltpu.make_async_copy(ids_hbm.at[pl.ds(0, NSEG)],
                                  ids_v.at[b], sem_i).wait()

        def gather_fire(b):
            for s in range(NSEG):
                pltpu.async_copy(table_hbm.at[ids_v.at[b, s]],
                                 rows_v.at[b, pl.ds(s * SEG, SEG)],
                                 sem_g[b])

        def gather_wait(b):
            pltpu.make_async_copy(table_hbm.at[pl.ds(0, NSEG * SEG)],
                                  rows_v.at[b], sem_g[b]).wait()

        def out_wait(b):
            pltpu.make_async_copy(sums_v.at[b],
                                  out_hbm.at[pl.ds(0, CB)], sem_o).wait()

        def reduce_and_out(c, b):
            rv = rows_v.at[b]
            sv = sums_v.at[b]
            for r in range(CB):
                def tok(t, acc):
                    i0 = r * SEQ + t * UNROLL
                    out = []
                    for g in range(NG):
                        sl = pl.ds(g * 16, 16)
                        out.append(acc[g]
                                   + ((rv[i0, sl] + rv[i0 + 1, sl])
                                      + (rv[i0 + 2, sl] + rv[i0 + 3, sl])))
                    return tuple(out)
                acc0 = tuple(jnp.zeros((16,), jnp.float32)
                             for _ in range(NG))
                acc = lax.fori_loop(0, SEQ // UNROLL, tok, acc0)
                for g in range(NG):
                    sv[r, pl.ds(g * 16, 16)] = acc[g]
            pltpu.async_copy(sv, out_hbm.at[pl.ds(row0 + c * CB, CB)], sem_o)

        ids_fire(0, 0)
        ids_wait(0)
        gather_fire(0)
        ids_fire(1, 1)

        def step(kk, carry):
            for b in range(2):
                c = 2 * kk + b
                gather_wait(b)

                @pl.when(c + 1 < NCHUNK)
                def _():
                    ids_wait(1 - b)
                    gather_fire(1 - b)

                @pl.when(c + 2 < NCHUNK)
                def _():
                    ids_fire(c + 2, b)

                @pl.when(c >= 2)
                def _():
                    out_wait(b)

                reduce_and_out(c, b)
            return carry

        lax.fori_loop(0, NCHUNK // 2, step, 0)
        out_wait(0)
        out_wait(1)

    return k(ids2d, table)


def _tc_pooler(sums, a, b):
    bt = 512

    def body(x_ref, a_ref, b_ref, o_ref):
        o_ref[...] = jnp.dot(x_ref[...], a_ref[...],
                             preferred_element_type=jnp.float32) + b_ref[...]

    return pl.pallas_call(
        body,
        grid=(BATCH // bt,),
        in_specs=[
            pl.BlockSpec((bt, HIDDEN), lambda i: (i, 0)),
            pl.BlockSpec((HIDDEN, HIDDEN), lambda i: (0, 0)),
            pl.BlockSpec((1, HIDDEN), lambda i: (0, 0)),
        ],
        out_specs=pl.BlockSpec((bt, HIDDEN), lambda i: (i, 0)),
        out_shape=jax.ShapeDtypeStruct((BATCH, HIDDEN), jnp.float32),
    )(sums, a, b)


def kernel(input_ids, embedding_table, pooler_w, pooler_b):
    ids2d = jnp.reshape(input_ids.astype(jnp.int32), (BATCH * SEQ // SEG, SEG))
    sums = _sc_pooled_sums(ids2d, embedding_table)
    a = pooler_w.T * (1.0 / SEQ)
    b2d = jnp.reshape(pooler_b, (1, HIDDEN))
    return _tc_pooler(sums, a, b2d)
